# Initial kernel scaffold; baseline (speedup 1.0000x reference)
import jax
import jax.numpy as jnp
from jax.experimental import pallas as pl


def kernel(inputs, tables, W1, b1, W2, b2):
    raise NotImplementedError("write your pallas kernel here")


# speedup vs baseline: 1.9813x; 1.9813x over previous
import functools

import jax
import jax.numpy as jnp
from jax import lax
from jax.experimental import pallas as pl
from jax.experimental.pallas import tpu as pltpu
from jax.experimental.pallas import tpu_sc as plsc

F = 26
CARD = 100000
E = 32
H = 32
O = 16
B = 4096
FB = F * B

NC = 2
NS = 16
NW = NC * NS
BPW = FB // NW
LANES = 16
NVEC = BPW // LANES
CHUNK = 128
NCHUNK = BPW // CHUNK

@functools.cache
def _make_sc_gather():
    mesh = plsc.VectorSubcoreMesh(core_axis_name="c", subcore_axis_name="s")
    return functools.partial(
        pl.kernel,
        mesh=mesh,
        compiler_params=pltpu.CompilerParams(use_tc_tiling_on_sc=False),
        out_type=jax.ShapeDtypeStruct((FB, E), jnp.float32),
        scratch_types=[
            pltpu.VMEM((BPW,), jnp.int32),
            pltpu.VMEM((BPW,), jnp.int32),
            pltpu.VMEM((BPW, E), jnp.float32),
            pltpu.SemaphoreType.DMA,
        ],
    )(_sc_gather_body)


def _sc_gather_body(idx_hbm, table_hbm, out_hbm, raw_v, idx_v, rows_v, sem):
    wid = lax.axis_index("s") * NC + lax.axis_index("c")
    base = wid * BPW

    pltpu.sync_copy(idx_hbm.at[pl.ds(base, BPW)], raw_v)

    def hash_body(i, carry):
        off = i * LANES
        x = raw_v[pl.ds(off, LANES)]
        pos = base + off + lax.broadcasted_iota(jnp.int32, (LANES,), 0)
        f = lax.shift_right_logical(pos, 12)
        y = lax.rem(x + 1, CARD)
        idx_v[pl.ds(off, LANES)] = f * CARD + y
        return carry

    lax.fori_loop(0, NVEC, hash_body, 0)

    copies = [
        pltpu.async_copy(
            table_hbm.at[idx_v.at[pl.ds(c * CHUNK, CHUNK)]],
            rows_v.at[pl.ds(c * CHUNK, CHUNK)],
            sem,
        )
        for c in range(NCHUNK)
    ]
    for cp in copies:
        cp.wait()

    pltpu.sync_copy(rows_v, out_hbm.at[pl.ds(base, BPW)])


BT = 1024
NB = B // BT


def _mlp_body(emb_ref, w1_ref, b1_ref, w2_ref, b2_ref, out_ref):
    e = emb_ref[0]
    h = jnp.dot(e, w1_ref[0], preferred_element_type=jnp.float32) + b1_ref[0]
    h = jnp.maximum(h, 0.0)
    out_ref[0] = jnp.dot(h, w2_ref[0], preferred_element_type=jnp.float32) + b2_ref[0]


_mlp = pl.pallas_call(
    _mlp_body,
    grid=(F, NB),
    in_specs=[
        pl.BlockSpec((1, BT, E), lambda f, b: (f, b, 0)),
        pl.BlockSpec((1, E, H), lambda f, b: (f, 0, 0)),
        pl.BlockSpec((1, 1, H), lambda f, b: (f, 0, 0)),
        pl.BlockSpec((1, H, O), lambda f, b: (f, 0, 0)),
        pl.BlockSpec((1, 1, O), lambda f, b: (f, 0, 0)),
    ],
    out_specs=pl.BlockSpec((1, BT, O), lambda f, b: (f, b, 0)),
    out_shape=jax.ShapeDtypeStruct((F, B, O), jnp.float32),
)


def kernel(inputs, tables, W1, b1, W2, b2):
    idx_flat = inputs.T.reshape(FB)
    table_flat = tables.reshape(F * CARD, E)
    emb = _make_sc_gather()(idx_flat, table_flat)
    emb3 = emb.reshape(F, B, E)
    out = _mlp(emb3, W1, b1.reshape(F, 1, H), W2, b2.reshape(F, 1, O))
    return out


# speedup vs baseline: 13.1820x; 6.6531x over previous
import functools

import jax
import jax.numpy as jnp
from jax import lax
from jax.experimental import pallas as pl
from jax.experimental.pallas import tpu as pltpu
from jax.experimental.pallas import tpu_sc as plsc

F = 26
CARD = 100000
E = 32
H = 32
O = 16
B = 4096
FB = F * B
ROWS = F * E

NC = 2
NS = 16
NW = NC * NS
NCHUNKS = 1
FC = F // NCHUNKS
CROWS = FC * E
RPW = CROWS // NW
LANES = 16
NVEC_B = B // LANES
NVEC_H = 2 * B // LANES


def _sc_gather_body(row_base, idx_hbm, table_hbm, out_hbm, idx2_v, row_v, out2_v, sem, sem_out):
    wid = lax.axis_index("s") * NC + lax.axis_index("c")
    r0 = row_base + wid * RPW
    f0 = r0 // E
    f1 = jnp.minimum(f0 + 1, F - 1)

    pltpu.sync_copy(idx_hbm.at[pl.ds(f0 * B, B)], idx2_v.at[pl.ds(0, B)])
    pltpu.sync_copy(idx_hbm.at[pl.ds(f1 * B, B)], idx2_v.at[pl.ds(B, B)])

    @plsc.parallel_loop(0, NVEC_H, unroll=4)
    def hash_body(i):
        off = i * LANES
        x = idx2_v[pl.ds(off, LANES)]
        idx2_v[pl.ds(off, LANES)] = lax.rem(x + 1, CARD)

    out_descs = [None, None]
    for k in range(RPW):
        r = r0 + k
        f = r // E
        selbase = (f - f0) * B
        pltpu.sync_copy(table_hbm.at[r], row_v)
        kk = k % 2
        if out_descs[kk] is not None:
            out_descs[kk].wait()

        @plsc.parallel_loop(0, NVEC_B, unroll=8)
        def gather_body(i):
            off = i * LANES
            idxs = idx2_v[pl.ds(selbase + off, LANES)]
            out2_v[pl.ds(kk * B + off, LANES)] = plsc.load_gather(row_v, [idxs])
        out_descs[kk] = pltpu.async_copy(
            out2_v.at[pl.ds(kk * B, B)], out_hbm.at[r - row_base], sem_out
        )
    for d in out_descs:
        if d is not None:
            d.wait()


@functools.cache
def _make_sc_gather(row_base):
    mesh = plsc.VectorSubcoreMesh(core_axis_name="c", subcore_axis_name="s")
    return functools.partial(
        pl.kernel,
        mesh=mesh,
        compiler_params=pltpu.CompilerParams(needs_layout_passes=False),
        out_type=jax.ShapeDtypeStruct((CROWS, B), jnp.float32),
        scratch_types=[
            pltpu.VMEM((2 * B,), jnp.int32),
            pltpu.VMEM((CARD,), jnp.float32),
            pltpu.VMEM((2 * B,), jnp.float32),
            pltpu.SemaphoreType.DMA,
            pltpu.SemaphoreType.DMA,
        ],
    )(functools.partial(_sc_gather_body, row_base))


BT = 4096
NB = B // BT


def _mlp_body(embT_ref, w1_ref, b1_ref, w2_ref, b2_ref, out_ref):
    eT = embT_ref[0]
    h = lax.dot_general(w1_ref[0], eT, (((0,), (0,)), ((), ())),
                        preferred_element_type=jnp.float32)
    h = jnp.maximum(h + b1_ref[0], 0.0)
    out_ref[0] = (
        lax.dot_general(w2_ref[0], h, (((0,), (0,)), ((), ())),
                        preferred_element_type=jnp.float32)
        + b2_ref[0]
    )


_mlp = pl.pallas_call(
    _mlp_body,
    grid=(FC, NB),
    in_specs=[
        pl.BlockSpec((1, E, BT), lambda f, b: (f, 0, b)),
        pl.BlockSpec((1, E, H), lambda f, b: (f, 0, 0)),
        pl.BlockSpec((1, H, 1), lambda f, b: (f, 0, 0)),
        pl.BlockSpec((1, H, O), lambda f, b: (f, 0, 0)),
        pl.BlockSpec((1, O, 1), lambda f, b: (f, 0, 0)),
    ],
    out_specs=pl.BlockSpec((1, O, BT), lambda f, b: (f, 0, b)),
    out_shape=jax.ShapeDtypeStruct((FC, O, B), jnp.float32),
)


def kernel(inputs, tables, W1, b1, W2, b2):
    idx_flat = inputs.T.reshape(FB)
    table_rows = jnp.swapaxes(tables, 1, 2).reshape(ROWS, CARD)
    outs = []
    for c in range(NCHUNKS):
        embT = _make_sc_gather(c * CROWS)(idx_flat, table_rows)
        embT3 = embT.reshape(FC, E, B)
        fsl = slice(c * FC, (c + 1) * FC)
        outs.append(
            _mlp(embT3, W1[fsl], b1[fsl].reshape(FC, H, 1),
                 W2[fsl], b2[fsl].reshape(FC, O, 1))
        )
    outT = jnp.concatenate(outs, axis=0)
    return jnp.swapaxes(outT, 1, 2)
